# baseline (speedup 1.0000x reference)
import functools

import jax
import jax.numpy as jnp
from jax import lax
from jax.experimental import pallas as pl
from jax.experimental.pallas import tpu as pltpu
from jax.experimental.pallas import tpu_sc as plsc

VOCAB = 100000
EMB = 128
CS = 64
CTX = 50
NEG = 10

ROWS = 1024
RPW = ROWS // 32
EROWS = 64
BLK = 20480
NBLK = 5
FROWS = NBLK * BLK // 128


def _sweep_body(idx_e_ref, e_any, pmt_ref, pst_ref, mw_ref, mbc_ref,
                uw_ref, ubc_ref, ww_ref, wbc_ref, a_ref, b_ref, mupv_ref,
                ev, sem):
    f32 = jnp.float32
    hi = jax.lax.Precision.HIGHEST
    i = pl.program_id(0)

    @pl.when(i == 0)
    def _mlp():
        def fire(j, carry):
            r = idx_e_ref[j]
            pltpu.make_async_copy(
                e_any.at[pl.ds(r, 1)], ev.at[pl.ds(j, 1)], sem).start()
            return carry
        lax.fori_loop(0, EROWS, fire, 0)
        pltpu.make_async_copy(e_any.at[pl.ds(0, EROWS)], ev, sem).wait()

        e = ev[...]
        r128 = lax.broadcasted_iota(jnp.int32, (EMB, EMB), 0)
        c128 = lax.broadcasted_iota(jnp.int32, (EMB, EMB), 1)
        ident = jnp.where(r128 == c128, 1.0, 0.0).astype(f32)
        ext = lax.dot_general(ident, e[0:8, :], (((1,), (1,)), ((), ())),
                              precision=hi, preferred_element_type=f32)
        ect = lax.dot_general(ident, e[8:64, :], (((1,), (1,)), ((), ())),
                              precision=hi, preferred_element_type=f32)
        ex_col = ext[:, 0:1]
        rw = jax.nn.relu(
            lax.dot_general(mw_ref[...], ex_col, (((1,), (0,)), ((), ())),
                            precision=hi, preferred_element_type=f32)
            + mbc_ref[...])
        rcm = jax.nn.relu(
            lax.dot_general(mw_ref[...], ect, (((1,), (0,)), ((), ())),
                            precision=hi, preferred_element_type=f32)
            + mbc_ref[...])
        ccol = lax.broadcasted_iota(jnp.int32, (CS, 56), 1)
        rcm = jnp.where(ccol < CTX, rcm, 0.0)
        h2 = jnp.sum(rcm, axis=1, keepdims=True)
        h = jnp.concatenate([CTX * rw, h2], axis=0)
        mu = lax.dot_general(uw_ref[...], h, (((1,), (0,)), ((), ())),
                             precision=hi, preferred_element_type=f32) \
            + ubc_ref[...]
        z = lax.dot_general(ww_ref[...], h, (((1,), (0,)), ((), ())),
                            precision=hi, preferred_element_type=f32) \
            + wbc_ref[...]
        pv = jax.nn.softplus(z)
        lpv = jnp.sum(jnp.log(pv))
        lane8 = lax.broadcasted_iota(jnp.int32, (CS, 8), 1)
        mupv_ref[...] = (
            jnp.where(lane8 == 0, jnp.broadcast_to(mu, (CS, 8)), 0.0)
            + jnp.where(lane8 == 1, jnp.broadcast_to(pv, (CS, 8)), 0.0)
            + jnp.where(lane8 == 2, lpv, 0.0))

    mu_col = mupv_ref[:, 0:1]
    pv_col = mupv_ref[:, 1:2]
    pm = pmt_ref[...]
    s = pst_ref[...]
    v = s * s
    d = pm - mu_col
    a_part = jnp.sum((pv_col + d * d) / v, axis=0, keepdims=True)
    p = s
    for half in (32, 16, 8, 4, 2, 1):
        p = p[0:half, :] * p[half:2 * half, :]
    b_part = 2.0 * jnp.log(p)
    a_ref[...] = jnp.concatenate(
        [a_part[:, k * 128:(k + 1) * 128] for k in range(BLK // 128)], axis=0)
    b_ref[...] = jnp.concatenate(
        [b_part[:, k * 128:(k + 1) * 128] for k in range(BLK // 128)], axis=0)


def _sweep(idx_e, E, pmT, psT, M_w, M_bc, U_w, U_bc, W_w, W_bc):
    last = VOCAB // BLK
    return pl.pallas_call(
        _sweep_body,
        grid=(NBLK,),
        in_specs=[
            pl.BlockSpec(memory_space=pltpu.SMEM),
            pl.BlockSpec(memory_space=pl.ANY),
            pl.BlockSpec((CS, BLK), lambda i: (0, jnp.minimum(i, last))),
            pl.BlockSpec((CS, BLK), lambda i: (0, jnp.minimum(i, last))),
            pl.BlockSpec((CS, EMB), lambda i: (0, 0)),
            pl.BlockSpec((CS, 1), lambda i: (0, 0)),
            pl.BlockSpec((CS, EMB), lambda i: (0, 0)),
            pl.BlockSpec((CS, 1), lambda i: (0, 0)),
            pl.BlockSpec((CS, EMB), lambda i: (0, 0)),
            pl.BlockSpec((CS, 1), lambda i: (0, 0)),
        ],
        out_specs=[
            pl.BlockSpec((BLK // 128, 128), lambda i: (i, 0)),
            pl.BlockSpec((BLK // 128, 128), lambda i: (i, 0)),
            pl.BlockSpec((CS, 8), lambda i: (0, 0)),
        ],
        out_shape=[
            jax.ShapeDtypeStruct((FROWS, 128), jnp.float32),
            jax.ShapeDtypeStruct((FROWS, 128), jnp.float32),
            jax.ShapeDtypeStruct((CS, 8), jnp.float32),
        ],
        scratch_shapes=[pltpu.VMEM((EROWS, EMB), jnp.float32),
                        pltpu.SemaphoreType.DMA],
    )(idx_e, E, pmT, psT, M_w, M_bc, U_w, U_bc, W_w, W_bc)


def _sc_gather(idx_all, a_arr, b_arr):
    mesh = plsc.VectorSubcoreMesh(core_axis_name="c", subcore_axis_name="s")

    @functools.partial(
        pl.kernel,
        out_type=(
            jax.ShapeDtypeStruct((ROWS, 128), jnp.float32),
            jax.ShapeDtypeStruct((ROWS, 128), jnp.float32),
        ),
        mesh=mesh,
        scratch_types=(
            pltpu.VMEM((RPW,), jnp.int32),
            pltpu.VMEM((RPW,), jnp.int32),
            pltpu.VMEM((RPW, 128), jnp.float32),
            pltpu.VMEM((RPW, 128), jnp.float32),
            pltpu.SemaphoreType.DMA,
            pltpu.SemaphoreType.DMA,
        ),
    )
    def k(idx_hbm, a_hbm, b_hbm, out_a, out_b, idxv, tilev, rows_a, rows_b,
          sem0, sem1):
        wid = lax.axis_index("s") * 2 + lax.axis_index("c")
        base = wid * RPW
        pltpu.sync_copy(idx_hbm.at[pl.ds(base, RPW)], idxv)
        for g in range(RPW // 16):
            iv = idxv[pl.ds(16 * g, 16)]
            tilev[pl.ds(16 * g, 16)] = lax.shift_right_logical(iv, 7)
        cp0 = pltpu.async_copy(a_hbm.at[tilev], rows_a, sem0)
        cp1 = pltpu.async_copy(b_hbm.at[tilev], rows_b, sem1)
        cp0.wait()
        pltpu.sync_copy(rows_a, out_a.at[pl.ds(base, RPW)])
        cp1.wait()
        pltpu.sync_copy(rows_b, out_b.at[pl.ds(base, RPW)])

    return k(idx_all, a_arr, b_arr)


def _hinge_body(ar_ref, br_ref, sub_ref, mupv_ref, out_ref):
    f32 = jnp.float32
    hi = jax.lax.Precision.HIGHEST
    lpv = mupv_ref[0:1, 2:3]
    lane = lax.broadcasted_iota(jnp.int32, (ROWS, 128), 1)
    onehot = jnp.where(lane == sub_ref[...], 1.0, 0.0).astype(f32)
    a = jnp.sum(ar_ref[...] * onehot, axis=1, keepdims=True)
    b = jnp.sum(br_ref[...] * onehot, axis=1, keepdims=True)
    kl = 0.5 * (a + b - CS - lpv)

    kl_x = kl[0:1, 0:1]
    kl_pos = kl[64:128, :]
    kl_neg = kl[256:768, :]
    irow = lax.broadcasted_iota(jnp.int32, (512, 64), 0)
    icol = lax.broadcasted_iota(jnp.int32, (512, 64), 1)
    sel = jnp.where(irow // NEG == icol, 1.0, 0.0).astype(f32)
    pos_for_neg = lax.dot_general(sel, kl_pos, (((1,), (0,)), ((), ())),
                                  precision=hi, preferred_element_type=f32)
    hinge = jnp.maximum(kl_neg - pos_for_neg + 1.0, 0.0)
    nrow = lax.broadcasted_iota(jnp.int32, (512, 1), 0)
    hinge = jnp.where(nrow < CTX * NEG, hinge, 0.0)
    out_ref[...] = jnp.sum(hinge, keepdims=True) - kl_x


def kernel(x, context, neg_samples, E, M_w, M_b, U_w, U_b, W_w, W_b,
           prior_mus, prior_sigmas):
    zi = jnp.zeros((), jnp.int32)
    x = x.astype(jnp.int32)
    context = context.astype(jnp.int32)
    negf = neg_samples.reshape(-1).astype(jnp.int32)
    idx_all = jnp.concatenate([
        x, jnp.full((63,), zi), context, jnp.full((142,), zi),
        negf, jnp.full((ROWS - 756,), zi)])
    idx_e = jnp.concatenate([x, jnp.full((7,), zi), context,
                             jnp.full((6,), zi)])
    sub = (idx_all & 127).reshape(ROWS, 1)

    a_arr, b_arr, mupv = _sweep(
        idx_e, E, prior_mus.T, prior_sigmas.T,
        M_w, M_b.reshape(CS, 1), U_w, U_b.reshape(CS, 1),
        W_w, W_b.reshape(CS, 1))
    ar, br = _sc_gather(idx_all, a_arr, b_arr)
    out = pl.pallas_call(
        _hinge_body,
        out_shape=jax.ShapeDtypeStruct((1, 1), jnp.float32),
    )(ar, br, sub, mupv)
    return out.reshape((1,))


# speedup vs baseline: 1.0137x; 1.0137x over previous
import functools

import jax
import jax.numpy as jnp
from jax import lax
from jax.experimental import pallas as pl
from jax.experimental.pallas import tpu as pltpu
from jax.experimental.pallas import tpu_sc as plsc

VOCAB = 100000
EMB = 128
CS = 64
CTX = 50
NEG = 10

ROWS = 1024
RPW = ROWS // 32
EROWS = 64
BLK = 10240
NBLK = 10
FROWS = NBLK * BLK // 128


def _sweep_body(idx_e_ref, e_any, pmt_ref, pst_ref, mw_ref, mbc_ref,
                uw_ref, ubc_ref, ww_ref, wbc_ref, a_ref, b_ref, mupv_ref,
                ev, sem):
    f32 = jnp.float32
    hi = jax.lax.Precision.HIGHEST
    i = pl.program_id(0)

    @pl.when(i == 0)
    def _mlp():
        def fire(j, carry):
            r = idx_e_ref[j]
            pltpu.make_async_copy(
                e_any.at[pl.ds(r, 1)], ev.at[pl.ds(j, 1)], sem).start()
            return carry
        lax.fori_loop(0, EROWS, fire, 0)
        pltpu.make_async_copy(e_any.at[pl.ds(0, EROWS)], ev, sem).wait()

        e = ev[...]
        r128 = lax.broadcasted_iota(jnp.int32, (EMB, EMB), 0)
        c128 = lax.broadcasted_iota(jnp.int32, (EMB, EMB), 1)
        ident = jnp.where(r128 == c128, 1.0, 0.0).astype(f32)
        ext = lax.dot_general(ident, e[0:8, :], (((1,), (1,)), ((), ())),
                              precision=hi, preferred_element_type=f32)
        ect = lax.dot_general(ident, e[8:64, :], (((1,), (1,)), ((), ())),
                              precision=hi, preferred_element_type=f32)
        ex_col = ext[:, 0:1]
        rw = jax.nn.relu(
            lax.dot_general(mw_ref[...], ex_col, (((1,), (0,)), ((), ())),
                            precision=hi, preferred_element_type=f32)
            + mbc_ref[...])
        rcm = jax.nn.relu(
            lax.dot_general(mw_ref[...], ect, (((1,), (0,)), ((), ())),
                            precision=hi, preferred_element_type=f32)
            + mbc_ref[...])
        ccol = lax.broadcasted_iota(jnp.int32, (CS, 56), 1)
        rcm = jnp.where(ccol < CTX, rcm, 0.0)
        h2 = jnp.sum(rcm, axis=1, keepdims=True)
        h = jnp.concatenate([CTX * rw, h2], axis=0)
        mu = lax.dot_general(uw_ref[...], h, (((1,), (0,)), ((), ())),
                             precision=hi, preferred_element_type=f32) \
            + ubc_ref[...]
        z = lax.dot_general(ww_ref[...], h, (((1,), (0,)), ((), ())),
                            precision=hi, preferred_element_type=f32) \
            + wbc_ref[...]
        pv = jax.nn.softplus(z)
        lpv = jnp.sum(jnp.log(pv))
        lane8 = lax.broadcasted_iota(jnp.int32, (CS, 8), 1)
        mupv_ref[...] = (
            jnp.where(lane8 == 0, jnp.broadcast_to(mu, (CS, 8)), 0.0)
            + jnp.where(lane8 == 1, jnp.broadcast_to(pv, (CS, 8)), 0.0)
            + jnp.where(lane8 == 2, lpv, 0.0))

    mu_col = mupv_ref[:, 0:1]
    pv_col = mupv_ref[:, 1:2]
    pm = pmt_ref[...]
    s = pst_ref[...]
    v = s * s
    d = pm - mu_col
    a_part = jnp.sum((pv_col + d * d) / v, axis=0, keepdims=True)
    p = s
    for half in (32, 16, 8, 4, 2, 1):
        p = p[0:half, :] * p[half:2 * half, :]
    b_part = 2.0 * jnp.log(p)
    a_ref[...] = jnp.concatenate(
        [a_part[:, k * 128:(k + 1) * 128] for k in range(BLK // 128)], axis=0)
    b_ref[...] = jnp.concatenate(
        [b_part[:, k * 128:(k + 1) * 128] for k in range(BLK // 128)], axis=0)


def _sweep(idx_e, E, pmT, psT, M_w, M_bc, U_w, U_bc, W_w, W_bc):
    last = VOCAB // BLK
    return pl.pallas_call(
        _sweep_body,
        grid=(NBLK,),
        in_specs=[
            pl.BlockSpec(memory_space=pltpu.SMEM),
            pl.BlockSpec(memory_space=pl.ANY),
            pl.BlockSpec((CS, BLK), lambda i: (0, jnp.minimum(i, last))),
            pl.BlockSpec((CS, BLK), lambda i: (0, jnp.minimum(i, last))),
            pl.BlockSpec((CS, EMB), lambda i: (0, 0)),
            pl.BlockSpec((CS, 1), lambda i: (0, 0)),
            pl.BlockSpec((CS, EMB), lambda i: (0, 0)),
            pl.BlockSpec((CS, 1), lambda i: (0, 0)),
            pl.BlockSpec((CS, EMB), lambda i: (0, 0)),
            pl.BlockSpec((CS, 1), lambda i: (0, 0)),
        ],
        out_specs=[
            pl.BlockSpec((BLK // 128, 128), lambda i: (i, 0)),
            pl.BlockSpec((BLK // 128, 128), lambda i: (i, 0)),
            pl.BlockSpec((CS, 8), lambda i: (0, 0)),
        ],
        out_shape=[
            jax.ShapeDtypeStruct((FROWS, 128), jnp.float32),
            jax.ShapeDtypeStruct((FROWS, 128), jnp.float32),
            jax.ShapeDtypeStruct((CS, 8), jnp.float32),
        ],
        scratch_shapes=[pltpu.VMEM((EROWS, EMB), jnp.float32),
                        pltpu.SemaphoreType.DMA],
    )(idx_e, E, pmT, psT, M_w, M_bc, U_w, U_bc, W_w, W_bc)


def _sc_gather(idx_all, a_arr, b_arr):
    mesh = plsc.VectorSubcoreMesh(core_axis_name="c", subcore_axis_name="s")

    @functools.partial(
        pl.kernel,
        out_type=(
            jax.ShapeDtypeStruct((ROWS, 128), jnp.float32),
            jax.ShapeDtypeStruct((ROWS, 128), jnp.float32),
        ),
        mesh=mesh,
        scratch_types=(
            pltpu.VMEM((RPW,), jnp.int32),
            pltpu.VMEM((RPW,), jnp.int32),
            pltpu.VMEM((RPW, 128), jnp.float32),
            pltpu.VMEM((RPW, 128), jnp.float32),
            pltpu.SemaphoreType.DMA,
            pltpu.SemaphoreType.DMA,
        ),
    )
    def k(idx_hbm, a_hbm, b_hbm, out_a, out_b, idxv, tilev, rows_a, rows_b,
          sem0, sem1):
        wid = lax.axis_index("s") * 2 + lax.axis_index("c")
        base = wid * RPW
        pltpu.sync_copy(idx_hbm.at[pl.ds(base, RPW)], idxv)
        for g in range(RPW // 16):
            iv = idxv[pl.ds(16 * g, 16)]
            tilev[pl.ds(16 * g, 16)] = lax.shift_right_logical(iv, 7)
        cp0 = pltpu.async_copy(a_hbm.at[tilev], rows_a, sem0)
        cp1 = pltpu.async_copy(b_hbm.at[tilev], rows_b, sem1)
        cp0.wait()
        pltpu.sync_copy(rows_a, out_a.at[pl.ds(base, RPW)])
        cp1.wait()
        pltpu.sync_copy(rows_b, out_b.at[pl.ds(base, RPW)])

    return k(idx_all, a_arr, b_arr)


def _hinge_body(ar_ref, br_ref, sub_ref, mupv_ref, out_ref):
    f32 = jnp.float32
    hi = jax.lax.Precision.HIGHEST
    lpv = mupv_ref[0:1, 2:3]
    lane = lax.broadcasted_iota(jnp.int32, (ROWS, 128), 1)
    onehot = jnp.where(lane == sub_ref[...], 1.0, 0.0).astype(f32)
    a = jnp.sum(ar_ref[...] * onehot, axis=1, keepdims=True)
    b = jnp.sum(br_ref[...] * onehot, axis=1, keepdims=True)
    kl = 0.5 * (a + b - CS - lpv)

    kl_x = kl[0:1, 0:1]
    kl_pos = kl[64:128, :]
    kl_neg = kl[256:768, :]
    irow = lax.broadcasted_iota(jnp.int32, (512, 64), 0)
    icol = lax.broadcasted_iota(jnp.int32, (512, 64), 1)
    sel = jnp.where(irow // NEG == icol, 1.0, 0.0).astype(f32)
    pos_for_neg = lax.dot_general(sel, kl_pos, (((1,), (0,)), ((), ())),
                                  precision=hi, preferred_element_type=f32)
    hinge = jnp.maximum(kl_neg - pos_for_neg + 1.0, 0.0)
    nrow = lax.broadcasted_iota(jnp.int32, (512, 1), 0)
    hinge = jnp.where(nrow < CTX * NEG, hinge, 0.0)
    out_ref[...] = jnp.sum(hinge, keepdims=True) - kl_x


def kernel(x, context, neg_samples, E, M_w, M_b, U_w, U_b, W_w, W_b,
           prior_mus, prior_sigmas):
    zi = jnp.zeros((), jnp.int32)
    x = x.astype(jnp.int32)
    context = context.astype(jnp.int32)
    negf = neg_samples.reshape(-1).astype(jnp.int32)
    idx_all = jnp.concatenate([
        x, jnp.full((63,), zi), context, jnp.full((142,), zi),
        negf, jnp.full((ROWS - 756,), zi)])
    idx_e = jnp.concatenate([x, jnp.full((7,), zi), context,
                             jnp.full((6,), zi)])
    sub = (idx_all & 127).reshape(ROWS, 1)

    a_arr, b_arr, mupv = _sweep(
        idx_e, E, prior_mus.T, prior_sigmas.T,
        M_w, M_b.reshape(CS, 1), U_w, U_b.reshape(CS, 1),
        W_w, W_b.reshape(CS, 1))
    ar, br = _sc_gather(idx_all, a_arr, b_arr)
    out = pl.pallas_call(
        _hinge_body,
        out_shape=jax.ShapeDtypeStruct((1, 1), jnp.float32),
    )(ar, br, sub, mupv)
    return out.reshape((1,))
